# baseline (speedup 1.0000x reference)
import functools
import math

import jax
import jax.numpy as jnp
from jax import lax
from jax.experimental import pallas as pl
from jax.experimental.pallas import tpu as pltpu
from jax.experimental.pallas import tpu_sc as plsc

N = 10000
NPAD = 10240
E = 320000
D = 128
OUT = 128
L = 8
ALPHA = 0.1
THETA = 0.5

CHUNK = 64
NSC = 2
NSUB = 16
NW = NSC * NSUB
EPW = 159
EPAD = NW * EPW * CHUNK
NBUF = 3
ROWS_PER_SUB = NPAD // NSUB
BR = 2000

_sc_mesh = plsc.VectorSubcoreMesh(core_axis_name="c", subcore_axis_name="s")


@functools.partial(
    pl.kernel,
    out_type=jax.ShapeDtypeStruct((NSC * NPAD, D), jnp.float32),
    mesh=_sc_mesh,
    scratch_types=[
        pltpu.VMEM((EPW, CHUNK), jnp.int32),
        pltpu.VMEM((NBUF, CHUNK), jnp.int32),
        pltpu.VMEM((NBUF, CHUNK), jnp.int32),
        pltpu.VMEM((NBUF, CHUNK, D), jnp.float32),
        pltpu.VMEM_SHARED((NPAD, D), jnp.float32),
        pltpu.SemaphoreType.DMA,
        pltpu.SemaphoreType.DMA,
        pltpu.SemaphoreType.DMA,
        pltpu.SemaphoreType.DMA,
        pltpu.SemaphoreType.DMA,
        pltpu.SemaphoreType.DMA,
        pltpu.SemaphoreType.DMA,
        pltpu.SemaphoreType.DMA,
        pltpu.SemaphoreType.DMA,
    ],
)
def _sc_propagate(xs_hbm, pk_hbm, out_hbm, pk_v, srcs, dsts, rows, acc,
                  g0, g1, g2, g3, s0, s1, s2, s3, zsem):
    gsem = (g0, g1, g2, g3)
    ssem = (s0, s1, s2, s3)
    cid = lax.axis_index("c")
    sid = lax.axis_index("s")
    wid = sid * NSC + cid

    pltpu.async_copy(pk_hbm.at[wid], pk_v, g3)

    def unpack(k, b):
        for j in range(CHUNK // 16):
            sl = pl.ds(j * 16, 16)
            p = pk_v[k, sl]
            srcs[b, sl] = p & 0xFFFF
            dsts[b, sl] = p >> 16

    def gather_start(k, b):
        pltpu.async_copy(xs_hbm.at[srcs.at[b]], rows.at[b], gsem[b])

    def gather_wait(b):
        pltpu.make_async_copy(xs_hbm.at[srcs.at[b]], rows.at[b],
                              gsem[b]).wait()

    def scatter_start(b):
        pltpu.async_copy(rows.at[b], acc.at[dsts.at[b]], ssem[b], add=True)

    def scatter_wait(b):
        pltpu.make_async_copy(rows.at[b], acc.at[dsts.at[b]], ssem[b]).wait()

    @pl.loop(0, CHUNK)
    def _(i):
        for j in range(D // 16):
            rows[0, i, pl.ds(j * 16, 16)] = jnp.zeros((16,), jnp.float32)

    base = sid * ROWS_PER_SUB
    nz = ROWS_PER_SUB // CHUNK
    for k in range(nz):
        pltpu.async_copy(rows.at[0], acc.at[pl.ds(base + k * CHUNK, CHUNK)],
                         zsem)
    pltpu.make_async_copy(pk_hbm.at[wid], pk_v, g3).wait()
    for b in range(NBUF):
        unpack(b, b)
    for b in range(1, NBUF):
        gather_start(b, b)
    for k in range(nz):
        pltpu.make_async_copy(rows.at[0], acc.at[pl.ds(base, CHUNK)],
                              zsem).wait()
    gather_start(0, 0)
    plsc.subcore_barrier()

    def step(k, b, first=False, fetch=True):
        bp = (b + NBUF - 1) % NBUF
        if not first:
            scatter_wait(bp)
            if fetch:
                unpack(k + NBUF - 1, bp)
                gather_start(k + NBUF - 1, bp)
        gather_wait(b)
        scatter_start(b)

    step(0, 0, first=True)
    for b in range(1, NBUF):
        step(b, b)

    @pl.loop(0, EPW // NBUF - 2)
    def _(i):
        for b in range(NBUF):
            k = (i + 1) * NBUF + b
            step(k, b)

    step(EPW - NBUF, 0)
    for b in range(1, NBUF):
        step(EPW - NBUF + b, b, fetch=False)
    scatter_wait(NBUF - 1)

    plsc.subcore_barrier()
    pltpu.sync_copy(acc.at[pl.ds(base, ROWS_PER_SUB)],
                    out_hbm.at[pl.ds(cid * NPAD + base, ROWS_PER_SUB)])


def _linear_in(x, w, b):
    def body(x_ref, w_ref, b_ref, o_ref):
        h = jnp.dot(x_ref[...], w_ref[...], preferred_element_type=jnp.float32)
        o_ref[...] = jnp.maximum(h + b_ref[...], 0.0)

    return pl.pallas_call(
        body,
        grid=(N // BR,),
        in_specs=[
            pl.BlockSpec((BR, D), lambda i: (i, 0)),
            pl.BlockSpec((D, D), lambda i: (0, 0)),
            pl.BlockSpec((1, D), lambda i: (0, 0)),
        ],
        out_specs=pl.BlockSpec((BR, D), lambda i: (i, 0)),
        out_shape=jax.ShapeDtypeStruct((NPAD, D), jnp.float32),
    )(x, w, b)


def _mix(agg, x0, w, beta):
    def body(agg_ref, x0_ref, w_ref, o_ref):
        a = agg_ref[0] + agg_ref[1]
        h = a * (1.0 - ALPHA) + ALPHA * x0_ref[...]
        hw = jnp.dot(h, w_ref[...], preferred_element_type=jnp.float32)
        o_ref[...] = jnp.maximum((1.0 - beta) * h + beta * hw, 0.0)

    return pl.pallas_call(
        body,
        grid=(N // BR,),
        in_specs=[
            pl.BlockSpec((NSC, BR, D), lambda i: (0, i, 0)),
            pl.BlockSpec((BR, D), lambda i: (i, 0)),
            pl.BlockSpec((D, D), lambda i: (0, 0)),
        ],
        out_specs=pl.BlockSpec((BR, D), lambda i: (i, 0)),
        out_shape=jax.ShapeDtypeStruct((NPAD, D), jnp.float32),
    )(agg, x0, w)


def _mix_out(agg, x0, w, beta, w2, b2):
    def body(agg_ref, x0_ref, w_ref, w2_ref, b2_ref, o_ref):
        a = agg_ref[0] + agg_ref[1]
        h = a * (1.0 - ALPHA) + ALPHA * x0_ref[...]
        hw = jnp.dot(h, w_ref[...], preferred_element_type=jnp.float32)
        r = jnp.maximum((1.0 - beta) * h + beta * hw, 0.0)
        o_ref[...] = jnp.dot(r, w2_ref[...],
                             preferred_element_type=jnp.float32) + b2_ref[...]

    return pl.pallas_call(
        body,
        grid=(N // BR,),
        in_specs=[
            pl.BlockSpec((NSC, BR, D), lambda i: (0, i, 0)),
            pl.BlockSpec((BR, D), lambda i: (i, 0)),
            pl.BlockSpec((D, D), lambda i: (0, 0)),
            pl.BlockSpec((D, OUT), lambda i: (0, 0)),
            pl.BlockSpec((1, OUT), lambda i: (0, 0)),
        ],
        out_specs=pl.BlockSpec((BR, OUT), lambda i: (i, 0)),
        out_shape=jax.ShapeDtypeStruct((N, OUT), jnp.float32),
    )(agg, x0, w, w2, b2)


def kernel(x, edge_index, W_in, b_in, W_convs, W_out, b_out):
    ei = edge_index.astype(jnp.int32)
    pad = N + (jnp.arange(EPAD - E, dtype=jnp.int32) % (NPAD - N))
    dst = jnp.concatenate([ei[0], pad])
    src = jnp.concatenate([ei[1], pad])
    packed = (src | (dst << 16)).reshape(NW, EPW, CHUNK)
    xs = _linear_in(x, W_in, b_in.reshape(1, D))
    x0 = xs
    for i in range(L - 1):
        beta = math.log(THETA / (i + 1) + 1.0)
        agg2 = _sc_propagate(xs, packed)
        xs = _mix(agg2.reshape(NSC, NPAD, D), x0, W_convs[i], beta)
    beta = math.log(THETA / L + 1.0)
    agg2 = _sc_propagate(xs, packed)
    return _mix_out(agg2.reshape(NSC, NPAD, D), x0, W_convs[L - 1], beta,
                    W_out, b_out.reshape(1, OUT))


# speedup vs baseline: 1.0168x; 1.0168x over previous
import functools
import math

import jax
import jax.numpy as jnp
from jax import lax
from jax.experimental import pallas as pl
from jax.experimental.pallas import tpu as pltpu
from jax.experimental.pallas import tpu_sc as plsc

N = 10000
NPAD = 10240
E = 320000
D = 128
OUT = 128
L = 8
ALPHA = 0.1
THETA = 0.5

CHUNK = 64
NSC = 2
NSUB = 16
NW = NSC * NSUB
EPW = 159
EPAD = NW * EPW * CHUNK
NBUF = 3
ROWS_PER_SUB = NPAD // NSUB
BR = 5000

_sc_mesh = plsc.VectorSubcoreMesh(core_axis_name="c", subcore_axis_name="s")


@functools.partial(
    pl.kernel,
    out_type=jax.ShapeDtypeStruct((NSC * NPAD, D), jnp.float32),
    mesh=_sc_mesh,
    scratch_types=[
        pltpu.VMEM((EPW, CHUNK), jnp.int32),
        pltpu.VMEM((NBUF, CHUNK), jnp.int32),
        pltpu.VMEM((NBUF, CHUNK), jnp.int32),
        pltpu.VMEM((NBUF, CHUNK, D), jnp.float32),
        pltpu.VMEM_SHARED((NPAD, D), jnp.float32),
        pltpu.SemaphoreType.DMA,
        pltpu.SemaphoreType.DMA,
        pltpu.SemaphoreType.DMA,
        pltpu.SemaphoreType.DMA,
        pltpu.SemaphoreType.DMA,
        pltpu.SemaphoreType.DMA,
        pltpu.SemaphoreType.DMA,
        pltpu.SemaphoreType.DMA,
        pltpu.SemaphoreType.DMA,
    ],
)
def _sc_propagate(xs_hbm, pk_hbm, out_hbm, pk_v, srcs, dsts, rows, acc,
                  g0, g1, g2, g3, s0, s1, s2, s3, zsem):
    gsem = (g0, g1, g2, g3)
    ssem = (s0, s1, s2, s3)
    cid = lax.axis_index("c")
    sid = lax.axis_index("s")
    wid = sid * NSC + cid

    pltpu.async_copy(pk_hbm.at[wid], pk_v, g3)

    def unpack(k, b):
        for j in range(CHUNK // 16):
            sl = pl.ds(j * 16, 16)
            p = pk_v[k, sl]
            srcs[b, sl] = p & 0xFFFF
            dsts[b, sl] = p >> 16

    def gather_start(k, b):
        pltpu.async_copy(xs_hbm.at[srcs.at[b]], rows.at[b], gsem[b])

    def gather_wait(b):
        pltpu.make_async_copy(xs_hbm.at[srcs.at[b]], rows.at[b],
                              gsem[b]).wait()

    def scatter_start(b):
        pltpu.async_copy(rows.at[b], acc.at[dsts.at[b]], ssem[b], add=True)

    def scatter_wait(b):
        pltpu.make_async_copy(rows.at[b], acc.at[dsts.at[b]], ssem[b]).wait()

    @pl.loop(0, CHUNK)
    def _(i):
        for j in range(D // 16):
            rows[0, i, pl.ds(j * 16, 16)] = jnp.zeros((16,), jnp.float32)

    base = sid * ROWS_PER_SUB
    nz = ROWS_PER_SUB // CHUNK
    for k in range(nz):
        pltpu.async_copy(rows.at[0], acc.at[pl.ds(base + k * CHUNK, CHUNK)],
                         zsem)
    pltpu.make_async_copy(pk_hbm.at[wid], pk_v, g3).wait()
    for b in range(NBUF):
        unpack(b, b)
    for b in range(1, NBUF):
        gather_start(b, b)
    for k in range(nz):
        pltpu.make_async_copy(rows.at[0], acc.at[pl.ds(base, CHUNK)],
                              zsem).wait()
    gather_start(0, 0)
    plsc.subcore_barrier()

    def step(k, b, first=False, fetch=True):
        bp = (b + NBUF - 1) % NBUF
        if not first:
            scatter_wait(bp)
            if fetch:
                unpack(k + NBUF - 1, bp)
                gather_start(k + NBUF - 1, bp)
        gather_wait(b)
        scatter_start(b)

    step(0, 0, first=True)
    for b in range(1, NBUF):
        step(b, b)

    @pl.loop(0, EPW // NBUF - 2)
    def _(i):
        for b in range(NBUF):
            k = (i + 1) * NBUF + b
            step(k, b)

    step(EPW - NBUF, 0)
    for b in range(1, NBUF):
        step(EPW - NBUF + b, b, fetch=False)
    scatter_wait(NBUF - 1)

    plsc.subcore_barrier()
    pltpu.sync_copy(acc.at[pl.ds(base, ROWS_PER_SUB)],
                    out_hbm.at[pl.ds(cid * NPAD + base, ROWS_PER_SUB)])


def _linear_in(x, w, b):
    def body(x_ref, w_ref, b_ref, o_ref):
        h = jnp.dot(x_ref[...], w_ref[...], preferred_element_type=jnp.float32)
        o_ref[...] = jnp.maximum(h + b_ref[...], 0.0)

    return pl.pallas_call(
        body,
        grid=(N // BR,),
        in_specs=[
            pl.BlockSpec((BR, D), lambda i: (i, 0)),
            pl.BlockSpec((D, D), lambda i: (0, 0)),
            pl.BlockSpec((1, D), lambda i: (0, 0)),
        ],
        out_specs=pl.BlockSpec((BR, D), lambda i: (i, 0)),
        out_shape=jax.ShapeDtypeStruct((NPAD, D), jnp.float32),
    )(x, w, b)


def _mix(agg, x0, w, beta):
    def body(agg_ref, x0_ref, w_ref, o_ref):
        a = agg_ref[0] + agg_ref[1]
        h = a * (1.0 - ALPHA) + ALPHA * x0_ref[...]
        hw = jnp.dot(h, w_ref[...], preferred_element_type=jnp.float32)
        o_ref[...] = jnp.maximum((1.0 - beta) * h + beta * hw, 0.0)

    return pl.pallas_call(
        body,
        grid=(N // BR,),
        in_specs=[
            pl.BlockSpec((NSC, BR, D), lambda i: (0, i, 0)),
            pl.BlockSpec((BR, D), lambda i: (i, 0)),
            pl.BlockSpec((D, D), lambda i: (0, 0)),
        ],
        out_specs=pl.BlockSpec((BR, D), lambda i: (i, 0)),
        out_shape=jax.ShapeDtypeStruct((NPAD, D), jnp.float32),
    )(agg, x0, w)


def _mix_out(agg, x0, w, beta, w2, b2):
    def body(agg_ref, x0_ref, w_ref, w2_ref, b2_ref, o_ref):
        a = agg_ref[0] + agg_ref[1]
        h = a * (1.0 - ALPHA) + ALPHA * x0_ref[...]
        hw = jnp.dot(h, w_ref[...], preferred_element_type=jnp.float32)
        r = jnp.maximum((1.0 - beta) * h + beta * hw, 0.0)
        o_ref[...] = jnp.dot(r, w2_ref[...],
                             preferred_element_type=jnp.float32) + b2_ref[...]

    return pl.pallas_call(
        body,
        grid=(N // BR,),
        in_specs=[
            pl.BlockSpec((NSC, BR, D), lambda i: (0, i, 0)),
            pl.BlockSpec((BR, D), lambda i: (i, 0)),
            pl.BlockSpec((D, D), lambda i: (0, 0)),
            pl.BlockSpec((D, OUT), lambda i: (0, 0)),
            pl.BlockSpec((1, OUT), lambda i: (0, 0)),
        ],
        out_specs=pl.BlockSpec((BR, OUT), lambda i: (i, 0)),
        out_shape=jax.ShapeDtypeStruct((N, OUT), jnp.float32),
    )(agg, x0, w, w2, b2)


def kernel(x, edge_index, W_in, b_in, W_convs, W_out, b_out):
    ei = edge_index.astype(jnp.int32)
    pad = N + (jnp.arange(EPAD - E, dtype=jnp.int32) % (NPAD - N))
    dst = jnp.concatenate([ei[0], pad])
    src = jnp.concatenate([ei[1], pad])
    packed = (src | (dst << 16)).reshape(NW, EPW, CHUNK)
    xs = _linear_in(x, W_in, b_in.reshape(1, D))
    x0 = xs
    for i in range(L - 1):
        beta = math.log(THETA / (i + 1) + 1.0)
        agg2 = _sc_propagate(xs, packed)
        xs = _mix(agg2.reshape(NSC, NPAD, D), x0, W_convs[i], beta)
    beta = math.log(THETA / L + 1.0)
    agg2 = _sc_propagate(xs, packed)
    return _mix_out(agg2.reshape(NSC, NPAD, D), x0, W_convs[L - 1], beta,
                    W_out, b_out.reshape(1, OUT))
